# baseline (speedup 1.0000x reference)
import functools

import jax
import jax.numpy as jnp
from jax.experimental import pallas as pl
from jax.experimental.pallas import tpu as pltpu


def _pass_a_body(x_ref, adj_ref, W1_ref, b1_ref, h_ref, adjb_ref, s1_ref):
    m = pl.program_id(0)

    @pl.when(m == 0)
    def _():
        s1_ref[...] = jnp.dot(x_ref[...], W1_ref[...],
                              preferred_element_type=jnp.float32
                              ).astype(jnp.bfloat16)

    ab = adj_ref[...].astype(jnp.bfloat16)
    adjb_ref[...] = ab
    acc = jnp.dot(ab, s1_ref[...], preferred_element_type=jnp.float32)
    h_ref[...] = jnp.maximum(acc + b1_ref[...], 0.0)


def _pass_b_body(h_ref, adjb_ref, W2_ref, b2_ref, out_ref, t_ref):
    m = pl.program_id(0)

    @pl.when(m == 0)
    def _():
        t_ref[...] = jnp.dot(h_ref[...], W2_ref[...],
                             preferred_element_type=jnp.float32
                             ).astype(jnp.bfloat16)

    o = jnp.dot(adjb_ref[...], t_ref[...],
                preferred_element_type=jnp.float32) + b2_ref[...]
    o = o - jnp.max(o, axis=1, keepdims=True)
    out_ref[...] = o - jnp.log(jnp.sum(jnp.exp(o), axis=1, keepdims=True))


def _pick_bm(n):
    for bm in (400, 200, 80, 40, 8):
        if n % bm == 0:
            return bm
    return n


@jax.jit
def kernel(x, adj, W1, b1, W2, b2):
    N, F = x.shape
    H = W1.shape[1]
    C = W2.shape[1]
    BM = _pick_bm(N)
    grid = (N // BM,)

    h, adjb = pl.pallas_call(
        _pass_a_body,
        grid=grid,
        in_specs=[
            pl.BlockSpec((N, F), lambda m: (0, 0)),
            pl.BlockSpec((BM, N), lambda m: (m, 0)),
            pl.BlockSpec((F, H), lambda m: (0, 0)),
            pl.BlockSpec((1, H), lambda m: (0, 0)),
        ],
        out_specs=[
            pl.BlockSpec((BM, H), lambda m: (m, 0)),
            pl.BlockSpec((BM, N), lambda m: (m, 0)),
        ],
        out_shape=[
            jax.ShapeDtypeStruct((N, H), jnp.float32),
            jax.ShapeDtypeStruct((N, N), jnp.bfloat16),
        ],
        scratch_shapes=[pltpu.VMEM((N, H), jnp.bfloat16)],
    )(x, adj, W1, b1.reshape(1, H))

    out = pl.pallas_call(
        _pass_b_body,
        grid=grid,
        in_specs=[
            pl.BlockSpec((N, H), lambda m: (0, 0)),
            pl.BlockSpec((BM, N), lambda m: (m, 0)),
            pl.BlockSpec((H, C), lambda m: (0, 0)),
            pl.BlockSpec((1, C), lambda m: (0, 0)),
        ],
        out_specs=pl.BlockSpec((BM, C), lambda m: (m, 0)),
        out_shape=jax.ShapeDtypeStruct((N, C), jnp.float32),
        scratch_shapes=[pltpu.VMEM((N, C), jnp.bfloat16)],
    )(h, adjb, W2, b2.reshape(1, C))
    return out


# speedup vs baseline: 1.1896x; 1.1896x over previous
import functools

import numpy as np

import jax
import jax.numpy as jnp
from jax.experimental import pallas as pl
from jax.experimental.pallas import tpu as pltpu

CH = 1792


def _s1_body(x_ref, W1_ref, s1_ref, st_ref, *, N, Npad, TW):
    full = jnp.dot(x_ref[...], W1_ref[...], preferred_element_type=jnp.float32)
    body_rows = N - TW
    s1_ref[pl.ds(0, body_rows), :] = full[:body_rows, :]
    if Npad > body_rows:
        s1_ref[pl.ds(body_rows, Npad - body_rows), :] = jnp.zeros(
            (Npad - body_rows, s1_ref.shape[1]), jnp.float32)
    if TW:
        st_ref[...] = full[body_rows:, :].astype(jnp.bfloat16)
    else:
        st_ref[...] = jnp.zeros_like(st_ref)


def _body(tro_ref, tco_ref, ncp_ref, ccls_ref, l1f_ref, l1_ref, l2_ref,
          re_ref, rel_ref, wo_ref, oi_ref, thr_ref,
          s1_ref, adj_hbm, atail_ref, stail_ref, W2_ref, b1_ref, b2_ref,
          out_ref, t_ref, ttail_ref, oacc_ref, hacc_ref, abuf_ref, sem_ref,
          *, n_steps, WL, TW, zero_buf):
    n = pl.program_id(0)
    slot = jax.lax.rem(n, 2)
    nslot = jax.lax.rem(n + 1, 2)

    def start_copy(i, s):
        ro = pl.multiple_of(tro_ref[i], 8)
        co = pl.multiple_of(tco_ref[i], 128)

        @pl.when(ccls_ref[i] == 0)
        def _():
            pltpu.make_async_copy(
                adj_hbm.at[pl.ds(ro, CH), pl.ds(co, CH)],
                abuf_ref.at[s], sem_ref.at[s]).start()

        @pl.when(ccls_ref[i] == 1)
        def _():
            pltpu.make_async_copy(
                adj_hbm.at[pl.ds(ro, CH), pl.ds(co, WL)],
                abuf_ref.at[s, :, pl.ds(0, WL)], sem_ref.at[s]).start()

    @pl.when(n == 0)
    def _():
        if zero_buf:
            abuf_ref[...] = jnp.zeros_like(abuf_ref)
        start_copy(0, 0)
        t_ref[...] = jnp.zeros_like(t_ref)
        ttail_ref[...] = jnp.zeros_like(ttail_ref)
        oacc_ref[...] = jnp.zeros_like(oacc_ref)

    @pl.when((n + 1 < n_steps) & (ncp_ref[n + 1] == 1))
    def _():
        start_copy(n + 1, nslot)

    tro = pl.multiple_of(tro_ref[n], 8)
    tco = pl.multiple_of(tco_ref[n], 128)

    @pl.when(ncp_ref[n] == 1)
    def _():
        @pl.when(ccls_ref[n] == 0)
        def _():
            pltpu.make_async_copy(
                adj_hbm.at[pl.ds(tro, CH), pl.ds(tco, CH)],
                abuf_ref.at[slot], sem_ref.at[slot]).wait()

        @pl.when(ccls_ref[n] == 1)
        def _():
            pltpu.make_async_copy(
                adj_hbm.at[pl.ds(tro, CH), pl.ds(tco, WL)],
                abuf_ref.at[slot, :, pl.ds(0, WL)], sem_ref.at[slot]).wait()

    aref = abuf_ref.at[slot]

    @pl.when(l1_ref[n] == 1)
    def _():
        contrib = jnp.dot(aref[...], s1_ref[pl.ds(tco, CH), :],
                          preferred_element_type=jnp.float32)

        @pl.when(l1f_ref[n] == 1)
        def _():
            hacc_ref[...] = contrib

        @pl.when(l1f_ref[n] == 0)
        def _():
            hacc_ref[...] += contrib

    def finish_row(split_store):
        tail = jnp.dot(atail_ref[pl.ds(tro, CH), :], stail_ref[...],
                       preferred_element_type=jnp.float32)
        h = jnp.maximum(hacc_ref[...] + tail + b1_ref[...], 0.0)
        tblk = jnp.dot(h, W2_ref[...], preferred_element_type=jnp.float32)
        if split_store:
            t_ref[pl.ds(tro, CH - TW), :] = tblk[:CH - TW, :]
            ttail_ref[...] = tblk[CH - TW:, :].astype(jnp.bfloat16)
        else:
            t_ref[pl.ds(tro, CH), :] = tblk

    @pl.when(re_ref[n] == 1)
    def _():
        finish_row(False)

    if TW:
        @pl.when(rel_ref[n] == 1)
        def _():
            finish_row(True)

    @pl.when(l2_ref[n] == 1)
    def _():
        contrib = jnp.dot(aref[...], t_ref[pl.ds(tco, CH), :],
                          preferred_element_type=jnp.float32)
        rows = jax.lax.broadcasted_iota(jnp.int32, contrib.shape, 0)
        contrib = jnp.where(rows >= thr_ref[n], contrib, 0.0)
        oacc_ref[pl.ds(tro, CH), :] += contrib

    @pl.when(wo_ref[n] == 1)
    def _():
        oi = pl.multiple_of(oi_ref[n] * CH, 8)
        tail = jnp.dot(atail_ref[pl.ds(oi, CH), :], ttail_ref[...],
                       preferred_element_type=jnp.float32)
        o = oacc_ref[pl.ds(oi, CH), :] + tail + b2_ref[...]
        o = o - jnp.max(o, axis=1, keepdims=True)
        out_ref[...] = o - jnp.log(jnp.sum(jnp.exp(o), axis=1, keepdims=True))


def _schedule(N):
    K = -(-N // CH)
    row_starts = [CH * r for r in range(K - 1)] + [N - CH]
    col_starts = [CH * c for c in range(K)]
    WL = (N - CH * (K - 1)) // 128 * 128
    TW = N - CH * (K - 1) - WL
    OV = CH * (K - 1) - (N - CH)
    steps = []
    for r in range(K):
        order = [c for c in range(K) if c != r] + [r]
        thr = OV if r == K - 1 else 0
        last_row = r == K - 1 and TW > 0
        for i, c in enumerate(order):
            diag = c == r
            steps.append((row_starts[r], col_starts[c], 1, int(c == K - 1),
                          int(i == 0), 1, int(c <= r),
                          int(diag and not last_row),
                          int(diag and last_row), 0, 0, thr))
    for r in range(K - 1):
        for c in range(r + 1, K):
            steps.append((row_starts[r], col_starts[c], 1, int(c == K - 1),
                          0, 0, 1, 0, 0, int(c == K - 1), r, 0))
    steps.append((row_starts[0], col_starts[0], 0, 0, 0, 0, 0, 0, 0, 1,
                  K - 1, 0))
    cols = [np.asarray(col, dtype=np.int32) for col in zip(*steps)]
    for i in (0, 1, 2, 3):
        cols[i] = np.concatenate([cols[i], cols[i][-1:] * 0])
    return cols, len(steps), K, WL, TW


@jax.jit
def kernel(x, adj, W1, b1, W2, b2):
    N, F = x.shape
    H = W1.shape[1]
    C = W2.shape[1]
    sched, n_steps, K, WL, TW = _schedule(N)
    Npad = CH * K
    TWp = TW if TW else 8

    s1, s1_tail = pl.pallas_call(
        functools.partial(_s1_body, N=N, Npad=Npad, TW=TW),
        grid=(1,),
        in_specs=[
            pl.BlockSpec((N, F), lambda i: (0, 0)),
            pl.BlockSpec((F, H), lambda i: (0, 0)),
        ],
        out_specs=[
            pl.BlockSpec((Npad, H), lambda i: (0, 0)),
            pl.BlockSpec((TWp, H), lambda i: (0, 0)),
        ],
        out_shape=[
            jax.ShapeDtypeStruct((Npad, H), jnp.float32),
            jax.ShapeDtypeStruct((TWp, H), jnp.bfloat16),
        ],
    )(x, W1)

    if TW:
        adj_tail = jnp.pad(adj[:, N - TW:].astype(jnp.bfloat16),
                           ((0, Npad - N), (0, 0)))
    else:
        adj_tail = jnp.zeros((Npad, TWp), jnp.bfloat16)

    grid_spec = pltpu.PrefetchScalarGridSpec(
        num_scalar_prefetch=12,
        grid=(n_steps,),
        in_specs=[
            pl.BlockSpec((Npad, H), lambda n, *s: (0, 0)),
            pl.BlockSpec(memory_space=pltpu.MemorySpace.HBM),
            pl.BlockSpec((Npad, TWp), lambda n, *s: (0, 0)),
            pl.BlockSpec((TWp, H), lambda n, *s: (0, 0)),
            pl.BlockSpec((H, C), lambda n, *s: (0, 0)),
            pl.BlockSpec((1, H), lambda n, *s: (0, 0)),
            pl.BlockSpec((1, C), lambda n, *s: (0, 0)),
        ],
        out_specs=pl.BlockSpec((CH, C), lambda n, *s: (s[10][n], 0)),
        scratch_shapes=[
            pltpu.VMEM((Npad, C), jnp.float32),
            pltpu.VMEM((TWp, C), jnp.bfloat16),
            pltpu.VMEM((Npad, C), jnp.float32),
            pltpu.VMEM((CH, H), jnp.float32),
            pltpu.VMEM((2, CH, CH), jnp.float32),
            pltpu.SemaphoreType.DMA((2,)),
        ],
    )

    zero_buf = bool(sched[3][0] or sched[3][1])

    out = pl.pallas_call(
        functools.partial(_body, n_steps=n_steps, WL=WL, TW=TW,
                          zero_buf=zero_buf),
        grid_spec=grid_spec,
        out_shape=jax.ShapeDtypeStruct((N, C), jnp.float32),
    )(*sched, s1, adj, adj_tail, s1_tail, W2, b1.reshape(1, H),
      b2.reshape(1, C))
    return out


# speedup vs baseline: 1.3079x; 1.0995x over previous
import functools

import numpy as np

import jax
import jax.numpy as jnp
from jax.experimental import pallas as pl
from jax.experimental.pallas import tpu as pltpu

CH = 2048


def _s1_body(x_ref, W1_ref, s1_ref, st_ref, *, N, Npad, TW):
    full = jnp.dot(x_ref[...], W1_ref[...], preferred_element_type=jnp.float32)
    body_rows = N - TW
    s1_ref[pl.ds(0, body_rows), :] = full[:body_rows, :]
    if Npad > body_rows:
        s1_ref[pl.ds(body_rows, Npad - body_rows), :] = jnp.zeros(
            (Npad - body_rows, s1_ref.shape[1]), jnp.float32)
    if TW:
        st_ref[...] = full[body_rows:, :].astype(jnp.bfloat16)
    else:
        st_ref[...] = jnp.zeros_like(st_ref)


def _body(tro_ref, tco_ref, ncp_ref, ccls_ref, l1f_ref, l1_ref, l2_ref,
          re_ref, rel_ref, wo_ref, oi_ref, thr_ref,
          s1_ref, adj_hbm, atail_ref, stail_ref, W2_ref, b1_ref, b2_ref,
          out_ref, t_ref, ttail_ref, oacc_ref, hacc_ref, abuf_ref, sem_ref,
          *, n_steps, WL, TW, zero_buf):
    n = pl.program_id(0)
    slot = jax.lax.rem(n, 2)
    nslot = jax.lax.rem(n + 1, 2)

    def start_copy(i, s):
        ro = pl.multiple_of(tro_ref[i], 8)
        co = pl.multiple_of(tco_ref[i], 128)

        @pl.when(ccls_ref[i] == 0)
        def _():
            pltpu.make_async_copy(
                adj_hbm.at[pl.ds(ro, CH), pl.ds(co, CH)],
                abuf_ref.at[s], sem_ref.at[s]).start()

        @pl.when(ccls_ref[i] == 1)
        def _():
            pltpu.make_async_copy(
                adj_hbm.at[pl.ds(ro, CH), pl.ds(co, WL)],
                abuf_ref.at[s, :, pl.ds(0, WL)], sem_ref.at[s]).start()

    @pl.when(n == 0)
    def _():
        if zero_buf:
            abuf_ref[...] = jnp.zeros_like(abuf_ref)
        start_copy(0, 0)
        t_ref[...] = jnp.zeros_like(t_ref)
        ttail_ref[...] = jnp.zeros_like(ttail_ref)
        oacc_ref[...] = jnp.zeros_like(oacc_ref)

    @pl.when((n + 1 < n_steps) & (ncp_ref[n + 1] == 1))
    def _():
        start_copy(n + 1, nslot)

    tro = pl.multiple_of(tro_ref[n], 8)
    tco = pl.multiple_of(tco_ref[n], 128)

    @pl.when(ncp_ref[n] == 1)
    def _():
        @pl.when(ccls_ref[n] == 0)
        def _():
            pltpu.make_async_copy(
                adj_hbm.at[pl.ds(tro, CH), pl.ds(tco, CH)],
                abuf_ref.at[slot], sem_ref.at[slot]).wait()

        @pl.when(ccls_ref[n] == 1)
        def _():
            pltpu.make_async_copy(
                adj_hbm.at[pl.ds(tro, CH), pl.ds(tco, WL)],
                abuf_ref.at[slot, :, pl.ds(0, WL)], sem_ref.at[slot]).wait()

    aref = abuf_ref.at[slot]

    @pl.when(l1_ref[n] == 1)
    def _():
        contrib = jnp.dot(aref[...], s1_ref[pl.ds(tco, CH), :],
                          preferred_element_type=jnp.float32)

        @pl.when(l1f_ref[n] == 1)
        def _():
            hacc_ref[...] = contrib

        @pl.when(l1f_ref[n] == 0)
        def _():
            hacc_ref[...] += contrib

    def finish_row(split_store):
        tail = jnp.dot(atail_ref[pl.ds(tro, CH), :], stail_ref[...],
                       preferred_element_type=jnp.float32)
        h = jnp.maximum(hacc_ref[...] + tail + b1_ref[...], 0.0)
        tblk = jnp.dot(h, W2_ref[...], preferred_element_type=jnp.float32)
        if split_store:
            t_ref[pl.ds(tro, CH - TW), :] = tblk[:CH - TW, :]
            ttail_ref[...] = tblk[CH - TW:, :].astype(jnp.bfloat16)
        else:
            t_ref[pl.ds(tro, CH), :] = tblk

    @pl.when(re_ref[n] == 1)
    def _():
        finish_row(False)

    if TW:
        @pl.when(rel_ref[n] == 1)
        def _():
            finish_row(True)

    @pl.when(l2_ref[n] == 1)
    def _():
        contrib = jnp.dot(aref[...], t_ref[pl.ds(tco, CH), :],
                          preferred_element_type=jnp.float32)
        rows = jax.lax.broadcasted_iota(jnp.int32, contrib.shape, 0)
        contrib = jnp.where(rows >= thr_ref[n], contrib, 0.0)
        oacc_ref[pl.ds(tro, CH), :] += contrib

    @pl.when(wo_ref[n] == 1)
    def _():
        oi = pl.multiple_of(oi_ref[n] * CH, 8)
        tail = jnp.dot(atail_ref[pl.ds(oi, CH), :], ttail_ref[...],
                       preferred_element_type=jnp.float32)
        o = oacc_ref[pl.ds(oi, CH), :] + tail + b2_ref[...]
        o = o - jnp.max(o, axis=1, keepdims=True)
        out_ref[...] = o - jnp.log(jnp.sum(jnp.exp(o), axis=1, keepdims=True))


def _schedule(N):
    K = -(-N // CH)
    row_starts = [CH * r for r in range(K - 1)] + [N - CH]
    col_starts = [CH * c for c in range(K)]
    WL = (N - CH * (K - 1)) // 128 * 128
    TW = N - CH * (K - 1) - WL
    OV = CH * (K - 1) - (N - CH)
    steps = []
    for r in range(K):
        order = [c for c in range(K) if c != r] + [r]
        thr = OV if r == K - 1 else 0
        last_row = r == K - 1 and TW > 0
        for i, c in enumerate(order):
            diag = c == r
            steps.append((row_starts[r], col_starts[c], 1, int(c == K - 1),
                          int(i == 0), 1, int(c <= r),
                          int(diag and not last_row),
                          int(diag and last_row), 0, 0, thr))
    for r in range(K - 1):
        for c in range(r + 1, K):
            steps.append((row_starts[r], col_starts[c], 1, int(c == K - 1),
                          0, 0, 1, 0, 0, int(c == K - 1), r, 0))
    steps.append((row_starts[0], col_starts[0], 0, 0, 0, 0, 0, 0, 0, 1,
                  K - 1, 0))
    cols = [np.asarray(col, dtype=np.int32) for col in zip(*steps)]
    for i in (0, 1, 2, 3):
        cols[i] = np.concatenate([cols[i], cols[i][-1:] * 0])
    return cols, len(steps), K, WL, TW


@jax.jit
def kernel(x, adj, W1, b1, W2, b2):
    N, F = x.shape
    H = W1.shape[1]
    C = W2.shape[1]
    sched, n_steps, K, WL, TW = _schedule(N)
    Npad = CH * K
    TWp = TW if TW else 8

    s1, s1_tail = pl.pallas_call(
        functools.partial(_s1_body, N=N, Npad=Npad, TW=TW),
        grid=(1,),
        in_specs=[
            pl.BlockSpec((N, F), lambda i: (0, 0)),
            pl.BlockSpec((F, H), lambda i: (0, 0)),
        ],
        out_specs=[
            pl.BlockSpec((Npad, H), lambda i: (0, 0)),
            pl.BlockSpec((TWp, H), lambda i: (0, 0)),
        ],
        out_shape=[
            jax.ShapeDtypeStruct((Npad, H), jnp.float32),
            jax.ShapeDtypeStruct((TWp, H), jnp.bfloat16),
        ],
    )(x, W1)

    if TW:
        adj_tail = jnp.pad(adj[:, N - TW:].astype(jnp.bfloat16),
                           ((0, Npad - N), (0, 0)))
    else:
        adj_tail = jnp.zeros((Npad, TWp), jnp.bfloat16)

    grid_spec = pltpu.PrefetchScalarGridSpec(
        num_scalar_prefetch=12,
        grid=(n_steps,),
        in_specs=[
            pl.BlockSpec((Npad, H), lambda n, *s: (0, 0)),
            pl.BlockSpec(memory_space=pltpu.MemorySpace.HBM),
            pl.BlockSpec((Npad, TWp), lambda n, *s: (0, 0)),
            pl.BlockSpec((TWp, H), lambda n, *s: (0, 0)),
            pl.BlockSpec((H, C), lambda n, *s: (0, 0)),
            pl.BlockSpec((1, H), lambda n, *s: (0, 0)),
            pl.BlockSpec((1, C), lambda n, *s: (0, 0)),
        ],
        out_specs=pl.BlockSpec((CH, C), lambda n, *s: (s[10][n], 0)),
        scratch_shapes=[
            pltpu.VMEM((Npad, C), jnp.float32),
            pltpu.VMEM((TWp, C), jnp.bfloat16),
            pltpu.VMEM((Npad, C), jnp.float32),
            pltpu.VMEM((CH, H), jnp.float32),
            pltpu.VMEM((2, CH, CH), jnp.float32),
            pltpu.SemaphoreType.DMA((2,)),
        ],
    )

    zero_buf = bool(sched[3][0] or sched[3][1])

    out = pl.pallas_call(
        functools.partial(_body, n_steps=n_steps, WL=WL, TW=TW,
                          zero_buf=zero_buf),
        grid_spec=grid_spec,
        out_shape=jax.ShapeDtypeStruct((N, C), jnp.float32),
    )(*sched, s1, adj, adj_tail, s1_tail, W2, b1.reshape(1, H),
      b2.reshape(1, C))
    return out


# speedup vs baseline: 1.3105x; 1.0020x over previous
import functools

import numpy as np

import jax
import jax.numpy as jnp
from jax.experimental import pallas as pl
from jax.experimental.pallas import tpu as pltpu

CH = 2048


def _s1_body(x_ref, W1_ref, s1_ref, st_ref, *, N, Npad, TW):
    full = jnp.dot(x_ref[...], W1_ref[...], preferred_element_type=jnp.float32)
    body_rows = N - TW
    s1_ref[pl.ds(0, body_rows), :] = full[:body_rows, :]
    if Npad > body_rows:
        s1_ref[pl.ds(body_rows, Npad - body_rows), :] = jnp.zeros(
            (Npad - body_rows, s1_ref.shape[1]), jnp.float32)
    if TW:
        st_ref[...] = full[body_rows:, :].astype(jnp.bfloat16)
    else:
        st_ref[...] = jnp.zeros_like(st_ref)


def _body(tro_ref, tco_ref, ncp_ref, ccls_ref, l1f_ref, l1_ref, l2_ref,
          re_ref, rel_ref, wo_ref, oi_ref, thr_ref,
          s1_ref, adj_hbm, atail_ref, stail_ref, W2_ref, b1_ref, b2_ref,
          out_ref, t_ref, ttail_ref, oacc_ref, hacc_ref, abuf_ref, sem_ref,
          *, n_steps, WL, TW, zero_buf):
    n = pl.program_id(0)
    slot = jax.lax.rem(n, 2)
    nslot = jax.lax.rem(n + 1, 2)

    HF = CH // 2

    def half_copies(i, s):
        ro = pl.multiple_of(tro_ref[i], 8)
        ro2 = pl.multiple_of(tro_ref[i] + HF, 8)
        co = pl.multiple_of(tco_ref[i], 128)
        wide = [
            pltpu.make_async_copy(
                adj_hbm.at[pl.ds(ro, HF), pl.ds(co, CH)],
                abuf_ref.at[s, pl.ds(0, HF), :], sem_ref.at[s, 0]),
            pltpu.make_async_copy(
                adj_hbm.at[pl.ds(ro2, HF), pl.ds(co, CH)],
                abuf_ref.at[s, pl.ds(HF, HF), :], sem_ref.at[s, 1]),
        ]
        narrow = [
            pltpu.make_async_copy(
                adj_hbm.at[pl.ds(ro, HF), pl.ds(co, WL)],
                abuf_ref.at[s, pl.ds(0, HF), pl.ds(0, WL)], sem_ref.at[s, 0]),
            pltpu.make_async_copy(
                adj_hbm.at[pl.ds(ro2, HF), pl.ds(co, WL)],
                abuf_ref.at[s, pl.ds(HF, HF), pl.ds(0, WL)],
                sem_ref.at[s, 1]),
        ]
        return wide, narrow

    def start_copy(i, s):
        wide, narrow = half_copies(i, s)

        @pl.when(ccls_ref[i] == 0)
        def _():
            wide[0].start()
            wide[1].start()

        @pl.when(ccls_ref[i] == 1)
        def _():
            narrow[0].start()
            narrow[1].start()

    @pl.when(n == 0)
    def _():
        if zero_buf:
            abuf_ref[...] = jnp.zeros_like(abuf_ref)
        start_copy(0, 0)
        t_ref[...] = jnp.zeros_like(t_ref)
        ttail_ref[...] = jnp.zeros_like(ttail_ref)
        oacc_ref[...] = jnp.zeros_like(oacc_ref)

    @pl.when((n + 1 < n_steps) & (ncp_ref[n + 1] == 1))
    def _():
        start_copy(n + 1, nslot)

    tro = pl.multiple_of(tro_ref[n], 8)
    tco = pl.multiple_of(tco_ref[n], 128)

    @pl.when(ncp_ref[n] == 1)
    def _():
        wide, narrow = half_copies(n, slot)

        @pl.when(ccls_ref[n] == 0)
        def _():
            wide[0].wait()
            wide[1].wait()

        @pl.when(ccls_ref[n] == 1)
        def _():
            narrow[0].wait()
            narrow[1].wait()

    aref = abuf_ref.at[slot]

    @pl.when(l1_ref[n] == 1)
    def _():
        contrib = jnp.dot(aref[...], s1_ref[pl.ds(tco, CH), :],
                          preferred_element_type=jnp.float32)

        @pl.when(l1f_ref[n] == 1)
        def _():
            hacc_ref[...] = contrib

        @pl.when(l1f_ref[n] == 0)
        def _():
            hacc_ref[...] += contrib

    def finish_row(split_store):
        tail = jnp.dot(atail_ref[pl.ds(tro, CH), :], stail_ref[...],
                       preferred_element_type=jnp.float32)
        h = jnp.maximum(hacc_ref[...] + tail + b1_ref[...], 0.0)
        tblk = jnp.dot(h, W2_ref[...], preferred_element_type=jnp.float32)
        if split_store:
            t_ref[pl.ds(tro, CH - TW), :] = tblk[:CH - TW, :]
            ttail_ref[...] = tblk[CH - TW:, :].astype(jnp.bfloat16)
        else:
            t_ref[pl.ds(tro, CH), :] = tblk

    @pl.when(re_ref[n] == 1)
    def _():
        finish_row(False)

    if TW:
        @pl.when(rel_ref[n] == 1)
        def _():
            finish_row(True)

    @pl.when(l2_ref[n] == 1)
    def _():
        contrib = jnp.dot(aref[...], t_ref[pl.ds(tco, CH), :],
                          preferred_element_type=jnp.float32)
        rows = jax.lax.broadcasted_iota(jnp.int32, contrib.shape, 0)
        contrib = jnp.where(rows >= thr_ref[n], contrib, 0.0)
        oacc_ref[pl.ds(tro, CH), :] += contrib

    @pl.when(wo_ref[n] == 1)
    def _():
        oi = pl.multiple_of(oi_ref[n] * CH, 8)
        tail = jnp.dot(atail_ref[pl.ds(oi, CH), :], ttail_ref[...],
                       preferred_element_type=jnp.float32)
        o = oacc_ref[pl.ds(oi, CH), :] + tail + b2_ref[...]
        o = o - jnp.max(o, axis=1, keepdims=True)
        out_ref[...] = o - jnp.log(jnp.sum(jnp.exp(o), axis=1, keepdims=True))


def _schedule(N):
    K = -(-N // CH)
    row_starts = [CH * r for r in range(K - 1)] + [N - CH]
    col_starts = [CH * c for c in range(K)]
    WL = (N - CH * (K - 1)) // 128 * 128
    TW = N - CH * (K - 1) - WL
    OV = CH * (K - 1) - (N - CH)
    steps = []
    for r in range(K):
        order = [c for c in range(K) if c != r] + [r]
        thr = OV if r == K - 1 else 0
        last_row = r == K - 1 and TW > 0
        for i, c in enumerate(order):
            diag = c == r
            steps.append((row_starts[r], col_starts[c], 1, int(c == K - 1),
                          int(i == 0), 1, int(c <= r),
                          int(diag and not last_row),
                          int(diag and last_row), 0, 0, thr))
    for r in range(K - 1):
        for c in range(r + 1, K):
            steps.append((row_starts[r], col_starts[c], 1, int(c == K - 1),
                          0, 0, 1, 0, 0, int(c == K - 1), r, 0))
    steps.append((row_starts[0], col_starts[0], 0, 0, 0, 0, 0, 0, 0, 1,
                  K - 1, 0))
    cols = [np.asarray(col, dtype=np.int32) for col in zip(*steps)]
    for i in (0, 1, 2, 3):
        cols[i] = np.concatenate([cols[i], cols[i][-1:] * 0])
    return cols, len(steps), K, WL, TW


@jax.jit
def kernel(x, adj, W1, b1, W2, b2):
    N, F = x.shape
    H = W1.shape[1]
    C = W2.shape[1]
    sched, n_steps, K, WL, TW = _schedule(N)
    Npad = CH * K
    TWp = TW if TW else 8

    s1, s1_tail = pl.pallas_call(
        functools.partial(_s1_body, N=N, Npad=Npad, TW=TW),
        grid=(1,),
        in_specs=[
            pl.BlockSpec((N, F), lambda i: (0, 0)),
            pl.BlockSpec((F, H), lambda i: (0, 0)),
        ],
        out_specs=[
            pl.BlockSpec((Npad, H), lambda i: (0, 0)),
            pl.BlockSpec((TWp, H), lambda i: (0, 0)),
        ],
        out_shape=[
            jax.ShapeDtypeStruct((Npad, H), jnp.float32),
            jax.ShapeDtypeStruct((TWp, H), jnp.bfloat16),
        ],
    )(x, W1)

    if TW:
        adj_tail = jnp.pad(adj[:, N - TW:].astype(jnp.bfloat16),
                           ((0, Npad - N), (0, 0)))
    else:
        adj_tail = jnp.zeros((Npad, TWp), jnp.bfloat16)

    grid_spec = pltpu.PrefetchScalarGridSpec(
        num_scalar_prefetch=12,
        grid=(n_steps,),
        in_specs=[
            pl.BlockSpec((Npad, H), lambda n, *s: (0, 0)),
            pl.BlockSpec(memory_space=pltpu.MemorySpace.HBM),
            pl.BlockSpec((Npad, TWp), lambda n, *s: (0, 0)),
            pl.BlockSpec((TWp, H), lambda n, *s: (0, 0)),
            pl.BlockSpec((H, C), lambda n, *s: (0, 0)),
            pl.BlockSpec((1, H), lambda n, *s: (0, 0)),
            pl.BlockSpec((1, C), lambda n, *s: (0, 0)),
        ],
        out_specs=pl.BlockSpec((CH, C), lambda n, *s: (s[10][n], 0)),
        scratch_shapes=[
            pltpu.VMEM((Npad, C), jnp.float32),
            pltpu.VMEM((TWp, C), jnp.bfloat16),
            pltpu.VMEM((Npad, C), jnp.float32),
            pltpu.VMEM((CH, H), jnp.float32),
            pltpu.VMEM((2, CH, CH), jnp.float32),
            pltpu.SemaphoreType.DMA((2, 2)),
        ],
    )

    zero_buf = bool(sched[3][0] or sched[3][1])

    out = pl.pallas_call(
        functools.partial(_body, n_steps=n_steps, WL=WL, TW=TW,
                          zero_buf=zero_buf),
        grid_spec=grid_spec,
        out_shape=jax.ShapeDtypeStruct((N, C), jnp.float32),
    )(*sched, s1, adj, adj_tail, s1_tail, W2, b1.reshape(1, H),
      b2.reshape(1, C))
    return out


# speedup vs baseline: 1.3318x; 1.0163x over previous
import functools

import numpy as np

import jax
import jax.numpy as jnp
from jax.experimental import pallas as pl
from jax.experimental.pallas import tpu as pltpu

CH = 2048


def _s1_body(x_ref, W1_ref, s1_ref, st_ref, *, N, Npad, TW):
    full = jnp.dot(x_ref[...], W1_ref[...], preferred_element_type=jnp.float32)
    body_rows = N - TW
    s1_ref[pl.ds(0, body_rows), :] = full[:body_rows, :]
    if Npad > body_rows:
        s1_ref[pl.ds(body_rows, Npad - body_rows), :] = jnp.zeros(
            (Npad - body_rows, s1_ref.shape[1]), jnp.float32)
    if TW:
        st_ref[...] = full[body_rows:, :].astype(jnp.bfloat16)
    else:
        st_ref[...] = jnp.zeros_like(st_ref)


def _body(tro_ref, tco_ref, ncp_ref, ccls_ref, l1f_ref, l1_ref, l2_ref,
          re_ref, rel_ref, wo_ref, oi_ref, thr_ref,
          s1_ref, adj_hbm, atail_ref, stail_ref, W2_ref, b1_ref, b2_ref,
          out_ref, st_ref, ttail_ref, oacc_ref, hacc_ref, abuf_ref, sem_ref,
          *, n_steps, WL, TW, zero_buf, H):
    n = pl.program_id(0)
    slot = jax.lax.rem(n, 2)
    nslot = jax.lax.rem(n + 1, 2)

    HF = CH // 2

    def half_copies(i, s):
        ro = pl.multiple_of(tro_ref[i], 8)
        ro2 = pl.multiple_of(tro_ref[i] + HF, 8)
        co = pl.multiple_of(tco_ref[i], 128)
        wide = [
            pltpu.make_async_copy(
                adj_hbm.at[pl.ds(ro, HF), pl.ds(co, CH)],
                abuf_ref.at[s, pl.ds(0, HF), :], sem_ref.at[s, 0]),
            pltpu.make_async_copy(
                adj_hbm.at[pl.ds(ro2, HF), pl.ds(co, CH)],
                abuf_ref.at[s, pl.ds(HF, HF), :], sem_ref.at[s, 1]),
        ]
        narrow = [
            pltpu.make_async_copy(
                adj_hbm.at[pl.ds(ro, HF), pl.ds(co, WL)],
                abuf_ref.at[s, pl.ds(0, HF), pl.ds(0, WL)], sem_ref.at[s, 0]),
            pltpu.make_async_copy(
                adj_hbm.at[pl.ds(ro2, HF), pl.ds(co, WL)],
                abuf_ref.at[s, pl.ds(HF, HF), pl.ds(0, WL)],
                sem_ref.at[s, 1]),
        ]
        return wide, narrow

    def start_copy(i, s):
        wide, narrow = half_copies(i, s)

        @pl.when(ccls_ref[i] == 0)
        def _():
            wide[0].start()
            wide[1].start()

        @pl.when(ccls_ref[i] == 1)
        def _():
            narrow[0].start()
            narrow[1].start()

    @pl.when(n == 0)
    def _():
        if zero_buf:
            abuf_ref[...] = jnp.zeros_like(abuf_ref)
        start_copy(0, 0)
        st_ref[:, :H] = s1_ref[...]
        st_ref[:, H:] = jnp.zeros_like(st_ref[:, H:])
        ttail_ref[...] = jnp.zeros_like(ttail_ref)
        oacc_ref[...] = jnp.zeros_like(oacc_ref)

    @pl.when((n + 1 < n_steps) & (ncp_ref[n + 1] == 1))
    def _():
        start_copy(n + 1, nslot)

    tro = pl.multiple_of(tro_ref[n], 8)
    tco = pl.multiple_of(tco_ref[n], 128)

    @pl.when(ncp_ref[n] == 1)
    def _():
        wide, narrow = half_copies(n, slot)

        @pl.when(ccls_ref[n] == 0)
        def _():
            wide[0].wait()
            wide[1].wait()

        @pl.when(ccls_ref[n] == 1)
        def _():
            narrow[0].wait()
            narrow[1].wait()

    aref = abuf_ref.at[slot]

    def l2_accum(contrib):
        rows = jax.lax.broadcasted_iota(jnp.int32, contrib.shape, 0)
        contrib = jnp.where(rows >= thr_ref[n], contrib, 0.0)
        oacc_ref[pl.ds(tro, CH), :] += contrib

    @pl.when((l1_ref[n] == 1) | (l2_ref[n] == 1))
    def _():
        contrib = jnp.dot(aref[...], st_ref[pl.ds(tco, CH), :],
                          preferred_element_type=jnp.float32)

        @pl.when(l1_ref[n] == 1)
        def _():
            @pl.when(l1f_ref[n] == 1)
            def _():
                hacc_ref[...] = contrib[:, :H]

            @pl.when(l1f_ref[n] == 0)
            def _():
                hacc_ref[...] += contrib[:, :H]

        @pl.when(l2_ref[n] == 1)
        def _():
            l2_accum(contrib[:, H:])

    def finish_row(split_store):
        tail = jnp.dot(atail_ref[pl.ds(tro, CH), :], stail_ref[...],
                       preferred_element_type=jnp.float32)
        h = jnp.maximum(hacc_ref[...] + tail + b1_ref[...], 0.0)
        tblk = jnp.dot(h, W2_ref[...], preferred_element_type=jnp.float32)
        if split_store:
            st_ref[pl.ds(tro, CH - TW), H:] = tblk[:CH - TW, :]
            ttail_ref[...] = tblk[CH - TW:, :].astype(jnp.bfloat16)
        else:
            st_ref[pl.ds(tro, CH), H:] = tblk
        l2_accum(jnp.dot(aref[...], st_ref[pl.ds(tco, CH), H:],
                         preferred_element_type=jnp.float32))

    @pl.when(re_ref[n] == 1)
    def _():
        finish_row(False)

    if TW:
        @pl.when(rel_ref[n] == 1)
        def _():
            finish_row(True)

    @pl.when(wo_ref[n] == 1)
    def _():
        oi = pl.multiple_of(oi_ref[n] * CH, 8)
        tail = jnp.dot(atail_ref[pl.ds(oi, CH), :], ttail_ref[...],
                       preferred_element_type=jnp.float32)
        o = oacc_ref[pl.ds(oi, CH), :] + tail + b2_ref[...]
        o = o - jnp.max(o, axis=1, keepdims=True)
        out_ref[...] = o - jnp.log(jnp.sum(jnp.exp(o), axis=1, keepdims=True))


def _schedule(N):
    K = -(-N // CH)
    row_starts = [CH * r for r in range(K - 1)] + [N - CH]
    col_starts = [CH * c for c in range(K)]
    WL = (N - CH * (K - 1)) // 128 * 128
    TW = N - CH * (K - 1) - WL
    OV = CH * (K - 1) - (N - CH)
    steps = []
    for r in range(K):
        order = [c for c in range(K) if c != r] + [r]
        thr = OV if r == K - 1 else 0
        last_row = r == K - 1 and TW > 0
        for i, c in enumerate(order):
            diag = c == r
            steps.append((row_starts[r], col_starts[c], 1, int(c == K - 1),
                          int(i == 0), 1, int(c < r),
                          int(diag and not last_row),
                          int(diag and last_row), 0, 0, thr))
    for r in range(K - 1):
        for c in range(r + 1, K):
            steps.append((row_starts[r], col_starts[c], 1, int(c == K - 1),
                          0, 0, 1, 0, 0, int(c == K - 1), r, 0))
    steps.append((row_starts[0], col_starts[0], 0, 0, 0, 0, 0, 0, 0, 1,
                  K - 1, 0))
    cols = [np.asarray(col, dtype=np.int32) for col in zip(*steps)]
    for i in (0, 1, 2, 3):
        cols[i] = np.concatenate([cols[i], cols[i][-1:] * 0])
    return cols, len(steps), K, WL, TW


@jax.jit
def kernel(x, adj, W1, b1, W2, b2):
    N, F = x.shape
    H = W1.shape[1]
    C = W2.shape[1]
    sched, n_steps, K, WL, TW = _schedule(N)
    Npad = CH * K
    TWp = TW if TW else 8

    s1, s1_tail = pl.pallas_call(
        functools.partial(_s1_body, N=N, Npad=Npad, TW=TW),
        grid=(1,),
        in_specs=[
            pl.BlockSpec((N, F), lambda i: (0, 0)),
            pl.BlockSpec((F, H), lambda i: (0, 0)),
        ],
        out_specs=[
            pl.BlockSpec((Npad, H), lambda i: (0, 0)),
            pl.BlockSpec((TWp, H), lambda i: (0, 0)),
        ],
        out_shape=[
            jax.ShapeDtypeStruct((Npad, H), jnp.float32),
            jax.ShapeDtypeStruct((TWp, H), jnp.bfloat16),
        ],
    )(x, W1)

    if TW:
        adj_tail = jnp.pad(adj[:, N - TW:].astype(jnp.bfloat16),
                           ((0, Npad - N), (0, 0)))
    else:
        adj_tail = jnp.zeros((Npad, TWp), jnp.bfloat16)

    grid_spec = pltpu.PrefetchScalarGridSpec(
        num_scalar_prefetch=12,
        grid=(n_steps,),
        in_specs=[
            pl.BlockSpec((Npad, H), lambda n, *s: (0, 0)),
            pl.BlockSpec(memory_space=pltpu.MemorySpace.HBM),
            pl.BlockSpec((Npad, TWp), lambda n, *s: (0, 0)),
            pl.BlockSpec((TWp, H), lambda n, *s: (0, 0)),
            pl.BlockSpec((H, C), lambda n, *s: (0, 0)),
            pl.BlockSpec((1, H), lambda n, *s: (0, 0)),
            pl.BlockSpec((1, C), lambda n, *s: (0, 0)),
        ],
        out_specs=pl.BlockSpec((CH, C), lambda n, *s: (s[10][n], 0)),
        scratch_shapes=[
            pltpu.VMEM((Npad, H + C), jnp.float32),
            pltpu.VMEM((TWp, C), jnp.bfloat16),
            pltpu.VMEM((Npad, C), jnp.float32),
            pltpu.VMEM((CH, H), jnp.float32),
            pltpu.VMEM((2, CH, CH), jnp.float32),
            pltpu.SemaphoreType.DMA((2, 2)),
        ],
    )

    zero_buf = bool(sched[3][0] or sched[3][1])

    out = pl.pallas_call(
        functools.partial(_body, n_steps=n_steps, WL=WL, TW=TW,
                          zero_buf=zero_buf, H=H),
        grid_spec=grid_spec,
        out_shape=jax.ShapeDtypeStruct((N, C), jnp.float32),
    )(*sched, s1, adj, adj_tail, s1_tail, W2, b1.reshape(1, H),
      b2.reshape(1, C))
    return out


# speedup vs baseline: 1.3704x; 1.0290x over previous
import functools

import numpy as np

import jax
import jax.numpy as jnp
from jax.experimental import pallas as pl
from jax.experimental.pallas import tpu as pltpu

CH = 2048


def _body(tro_ref, tco_ref, ncp_ref, ccls_ref, l1f_ref, l1_ref, l2_ref,
          re_ref, rel_ref, wo_ref, oi_ref, thr_ref,
          x_ref, adj_hbm, atail_ref, W1_ref, W2_ref, b1_ref, b2_ref,
          out_ref, st_ref, stail_ref, ttail_ref, oacc_ref, hacc_ref,
          abuf_ref, sem_ref,
          *, n_steps, WL, TW, zero_buf, N, Npad, H):
    n = pl.program_id(0)
    slot = jax.lax.rem(n, 2)
    nslot = jax.lax.rem(n + 1, 2)

    HF = CH // 2

    def half_copies(i, s):
        ro = pl.multiple_of(tro_ref[i], 8)
        ro2 = pl.multiple_of(tro_ref[i] + HF, 8)
        co = pl.multiple_of(tco_ref[i], 128)
        wide = [
            pltpu.make_async_copy(
                adj_hbm.at[pl.ds(ro, HF), pl.ds(co, CH)],
                abuf_ref.at[s, pl.ds(0, HF), :], sem_ref.at[s, 0]),
            pltpu.make_async_copy(
                adj_hbm.at[pl.ds(ro2, HF), pl.ds(co, CH)],
                abuf_ref.at[s, pl.ds(HF, HF), :], sem_ref.at[s, 1]),
        ]
        narrow = [
            pltpu.make_async_copy(
                adj_hbm.at[pl.ds(ro, HF), pl.ds(co, WL)],
                abuf_ref.at[s, pl.ds(0, HF), pl.ds(0, WL)], sem_ref.at[s, 0]),
            pltpu.make_async_copy(
                adj_hbm.at[pl.ds(ro2, HF), pl.ds(co, WL)],
                abuf_ref.at[s, pl.ds(HF, HF), pl.ds(0, WL)],
                sem_ref.at[s, 1]),
        ]
        return wide, narrow

    def start_copy(i, s):
        wide, narrow = half_copies(i, s)

        @pl.when(ccls_ref[i] == 0)
        def _():
            wide[0].start()
            wide[1].start()

        @pl.when(ccls_ref[i] == 1)
        def _():
            narrow[0].start()
            narrow[1].start()

    @pl.when(n == 0)
    def _():
        if zero_buf:
            abuf_ref[...] = jnp.zeros_like(abuf_ref)
        start_copy(0, 0)
        full = jnp.dot(x_ref[...], W1_ref[...],
                       preferred_element_type=jnp.float32)
        body_rows = N - TW
        st_ref[pl.ds(0, body_rows), :H] = full[:body_rows, :]
        if Npad > body_rows:
            st_ref[pl.ds(body_rows, Npad - body_rows), :H] = jnp.zeros(
                (Npad - body_rows, H), jnp.float32)
        st_ref[:, H:] = jnp.zeros_like(st_ref[:, H:])
        if TW:
            stail_ref[...] = full[body_rows:, :].astype(jnp.bfloat16)
        else:
            stail_ref[...] = jnp.zeros_like(stail_ref)
        ttail_ref[...] = jnp.zeros_like(ttail_ref)
        oacc_ref[...] = jnp.zeros_like(oacc_ref)

    @pl.when((n + 1 < n_steps) & (ncp_ref[n + 1] == 1))
    def _():
        start_copy(n + 1, nslot)

    tro = pl.multiple_of(tro_ref[n], 8)
    tco = pl.multiple_of(tco_ref[n], 128)

    @pl.when(ncp_ref[n] == 1)
    def _():
        wide, narrow = half_copies(n, slot)

        @pl.when(ccls_ref[n] == 0)
        def _():
            wide[0].wait()
            wide[1].wait()

        @pl.when(ccls_ref[n] == 1)
        def _():
            narrow[0].wait()
            narrow[1].wait()

    aref = abuf_ref.at[slot]

    def l2_accum(contrib):
        rows = jax.lax.broadcasted_iota(jnp.int32, contrib.shape, 0)
        contrib = jnp.where(rows >= thr_ref[n], contrib, 0.0)
        oacc_ref[pl.ds(tro, CH), :] += contrib

    @pl.when((l1_ref[n] == 1) | (l2_ref[n] == 1))
    def _():
        contrib = jnp.dot(aref[...], st_ref[pl.ds(tco, CH), :],
                          preferred_element_type=jnp.float32)

        @pl.when(l1_ref[n] == 1)
        def _():
            @pl.when(l1f_ref[n] == 1)
            def _():
                hacc_ref[...] = contrib[:, :H]

            @pl.when(l1f_ref[n] == 0)
            def _():
                hacc_ref[...] += contrib[:, :H]

        @pl.when(l2_ref[n] == 1)
        def _():
            l2_accum(contrib[:, H:])

    def finish_row(split_store):
        tail = jnp.dot(atail_ref[pl.ds(tro, CH), :], stail_ref[...],
                       preferred_element_type=jnp.float32)
        h = jnp.maximum(hacc_ref[...] + tail + b1_ref[...], 0.0)
        tblk = jnp.dot(h, W2_ref[...], preferred_element_type=jnp.float32)
        if split_store:
            st_ref[pl.ds(tro, CH - TW), H:] = tblk[:CH - TW, :]
            ttail_ref[...] = tblk[CH - TW:, :].astype(jnp.bfloat16)
        else:
            st_ref[pl.ds(tro, CH), H:] = tblk
        l2_accum(jnp.dot(aref[...], st_ref[pl.ds(tco, CH), H:],
                         preferred_element_type=jnp.float32))

    @pl.when(re_ref[n] == 1)
    def _():
        finish_row(False)

    if TW:
        @pl.when(rel_ref[n] == 1)
        def _():
            finish_row(True)

    @pl.when(wo_ref[n] == 1)
    def _():
        oi = pl.multiple_of(oi_ref[n] * CH, 8)
        tail = jnp.dot(atail_ref[pl.ds(oi, CH), :], ttail_ref[...],
                       preferred_element_type=jnp.float32)
        o = oacc_ref[pl.ds(oi, CH), :] + tail + b2_ref[...]
        o = o - jnp.max(o, axis=1, keepdims=True)
        out_ref[...] = o - jnp.log(jnp.sum(jnp.exp(o), axis=1, keepdims=True))


def _schedule(N):
    K = -(-N // CH)
    row_starts = [CH * r for r in range(K - 1)] + [N - CH]
    col_starts = [CH * c for c in range(K)]
    WL = (N - CH * (K - 1)) // 128 * 128
    TW = N - CH * (K - 1) - WL
    OV = CH * (K - 1) - (N - CH)
    steps = []
    for r in range(K):
        order = [c for c in range(K) if c != r] + [r]
        thr = OV if r == K - 1 else 0
        last_row = r == K - 1 and TW > 0
        for i, c in enumerate(order):
            diag = c == r
            steps.append((row_starts[r], col_starts[c], 1, int(c == K - 1),
                          int(i == 0), 1, int(c < r),
                          int(diag and not last_row),
                          int(diag and last_row), 0, 0, thr))
    for r in range(K - 1):
        for c in range(r + 1, K):
            steps.append((row_starts[r], col_starts[c], 1, int(c == K - 1),
                          0, 0, 1, 0, 0, int(c == K - 1), r, 0))
    steps.append((row_starts[0], col_starts[0], 0, 0, 0, 0, 0, 0, 0, 1,
                  K - 1, 0))
    cols = [np.asarray(col, dtype=np.int32) for col in zip(*steps)]
    for i in (0, 1, 2, 3):
        cols[i] = np.concatenate([cols[i], cols[i][-1:] * 0])
    return cols, len(steps), K, WL, TW


@jax.jit
def kernel(x, adj, W1, b1, W2, b2):
    N, F = x.shape
    H = W1.shape[1]
    C = W2.shape[1]
    sched, n_steps, K, WL, TW = _schedule(N)
    Npad = CH * K
    TWp = TW if TW else 8

    if TW:
        adj_tail = jnp.pad(adj[:, N - TW:].astype(jnp.bfloat16),
                           ((0, Npad - N), (0, 0)))
    else:
        adj_tail = jnp.zeros((Npad, TWp), jnp.bfloat16)

    grid_spec = pltpu.PrefetchScalarGridSpec(
        num_scalar_prefetch=12,
        grid=(n_steps,),
        in_specs=[
            pl.BlockSpec((N, F), lambda n, *s: (0, 0)),
            pl.BlockSpec(memory_space=pltpu.MemorySpace.HBM),
            pl.BlockSpec((Npad, TWp), lambda n, *s: (0, 0)),
            pl.BlockSpec((F, H), lambda n, *s: (0, 0)),
            pl.BlockSpec((H, C), lambda n, *s: (0, 0)),
            pl.BlockSpec((1, H), lambda n, *s: (0, 0)),
            pl.BlockSpec((1, C), lambda n, *s: (0, 0)),
        ],
        out_specs=pl.BlockSpec((CH, C), lambda n, *s: (s[10][n], 0)),
        scratch_shapes=[
            pltpu.VMEM((Npad, H + C), jnp.float32),
            pltpu.VMEM((TWp, H), jnp.bfloat16),
            pltpu.VMEM((TWp, C), jnp.bfloat16),
            pltpu.VMEM((Npad, C), jnp.float32),
            pltpu.VMEM((CH, H), jnp.float32),
            pltpu.VMEM((2, CH, CH), jnp.float32),
            pltpu.SemaphoreType.DMA((2, 2)),
        ],
    )

    zero_buf = bool(sched[3][0] or sched[3][1])

    out = pl.pallas_call(
        functools.partial(_body, n_steps=n_steps, WL=WL, TW=TW,
                          zero_buf=zero_buf, N=N, Npad=Npad, H=H),
        grid_spec=grid_spec,
        out_shape=jax.ShapeDtypeStruct((N, C), jnp.float32),
    )(*sched, x, adj, adj_tail, W1, W2, b1.reshape(1, H), b2.reshape(1, C))
    return out


# speedup vs baseline: 1.3708x; 1.0003x over previous
import functools

import numpy as np

import jax
import jax.numpy as jnp
from jax.experimental import pallas as pl
from jax.experimental.pallas import tpu as pltpu

CH = 2048


def _body(tro_ref, tco_ref, ncp_ref, ccls_ref, l1f_ref, l1_ref, l2_ref,
          re_ref, rel_ref, wo_ref, oi_ref, thr_ref,
          x_ref, adj_hbm, atail_ref, W1_ref, W2_ref, b1_ref, b2_ref,
          out_ref, st_ref, stail_ref, oacc_ref, hacc_ref,
          abuf_ref, sem_ref,
          *, n_steps, WL, TW, zero_buf, N, Npad, H):
    n = pl.program_id(0)
    slot = jax.lax.rem(n, 2)
    nslot = jax.lax.rem(n + 1, 2)

    HF = CH // 2

    def half_copies(i, s):
        ro = pl.multiple_of(tro_ref[i], 8)
        ro2 = pl.multiple_of(tro_ref[i] + HF, 8)
        co = pl.multiple_of(tco_ref[i], 128)
        wide = [
            pltpu.make_async_copy(
                adj_hbm.at[pl.ds(ro, HF), pl.ds(co, CH)],
                abuf_ref.at[s, pl.ds(0, HF), :], sem_ref.at[s, 0]),
            pltpu.make_async_copy(
                adj_hbm.at[pl.ds(ro2, HF), pl.ds(co, CH)],
                abuf_ref.at[s, pl.ds(HF, HF), :], sem_ref.at[s, 1]),
        ]
        narrow = [
            pltpu.make_async_copy(
                adj_hbm.at[pl.ds(ro, HF), pl.ds(co, WL)],
                abuf_ref.at[s, pl.ds(0, HF), pl.ds(0, WL)], sem_ref.at[s, 0]),
            pltpu.make_async_copy(
                adj_hbm.at[pl.ds(ro2, HF), pl.ds(co, WL)],
                abuf_ref.at[s, pl.ds(HF, HF), pl.ds(0, WL)],
                sem_ref.at[s, 1]),
        ]
        return wide, narrow

    def start_copy(i, s):
        wide, narrow = half_copies(i, s)

        @pl.when(ccls_ref[i] == 0)
        def _():
            wide[0].start()
            wide[1].start()

        @pl.when(ccls_ref[i] == 1)
        def _():
            narrow[0].start()
            narrow[1].start()

    @pl.when(n == 0)
    def _():
        if zero_buf:
            abuf_ref[...] = jnp.zeros_like(abuf_ref)
        start_copy(0, 0)
        full = jnp.dot(x_ref[...], W1_ref[...],
                       preferred_element_type=jnp.float32)
        body_rows = N - TW
        st_ref[pl.ds(0, body_rows), :H] = full[:body_rows, :]
        if Npad > body_rows:
            st_ref[pl.ds(body_rows, Npad - body_rows), :H] = jnp.zeros(
                (Npad - body_rows, H), jnp.float32)
        st_ref[:, H:] = jnp.zeros_like(st_ref[:, H:])
        if TW:
            stail_ref[...] = full[body_rows:, :].astype(jnp.bfloat16)
        else:
            stail_ref[...] = jnp.zeros_like(stail_ref)
        oacc_ref[...] = jnp.zeros_like(oacc_ref)

    @pl.when((n + 1 < n_steps) & (ncp_ref[n + 1] == 1))
    def _():
        start_copy(n + 1, nslot)

    tro = pl.multiple_of(tro_ref[n], 8)
    tco = pl.multiple_of(tco_ref[n], 128)

    @pl.when(ncp_ref[n] == 1)
    def _():
        wide, narrow = half_copies(n, slot)

        @pl.when(ccls_ref[n] == 0)
        def _():
            wide[0].wait()
            wide[1].wait()

        @pl.when(ccls_ref[n] == 1)
        def _():
            narrow[0].wait()
            narrow[1].wait()

    aref = abuf_ref.at[slot]

    def l2_accum(contrib):
        rows = jax.lax.broadcasted_iota(jnp.int32, contrib.shape, 0)
        contrib = jnp.where(rows >= thr_ref[n], contrib, 0.0)
        oacc_ref[pl.ds(tro, CH), :] += contrib

    @pl.when((l1_ref[n] == 1) | (l2_ref[n] == 1))
    def _():
        contrib = jnp.dot(aref[...], st_ref[pl.ds(tco, CH), :],
                          preferred_element_type=jnp.float32)

        @pl.when(l1_ref[n] == 1)
        def _():
            @pl.when(l1f_ref[n] == 1)
            def _():
                hacc_ref[...] = contrib[:, :H]

            @pl.when(l1f_ref[n] == 0)
            def _():
                hacc_ref[...] += contrib[:, :H]

        @pl.when(l2_ref[n] == 1)
        def _():
            l2_accum(contrib[:, H:])

    def finish_row(split_store):
        tail = jnp.dot(atail_ref[pl.ds(tro, CH), :], stail_ref[...],
                       preferred_element_type=jnp.float32)
        h = jnp.maximum(hacc_ref[...] + tail + b1_ref[...], 0.0)
        tblk = jnp.dot(h, W2_ref[...], preferred_element_type=jnp.float32)
        if split_store:
            st_ref[pl.ds(tro, CH - TW), H:] = tblk[:CH - TW, :]
            ttail = tblk[CH - TW:, :].astype(jnp.bfloat16)
            oacc_ref[pl.ds(0, N), :] += jnp.dot(
                atail_ref[...], ttail, preferred_element_type=jnp.float32)
        else:
            st_ref[pl.ds(tro, CH), H:] = tblk
        l2_accum(jnp.dot(aref[...], st_ref[pl.ds(tco, CH), H:],
                         preferred_element_type=jnp.float32))

    @pl.when(re_ref[n] == 1)
    def _():
        finish_row(False)

    if TW:
        @pl.when(rel_ref[n] == 1)
        def _():
            finish_row(True)

    @pl.when(wo_ref[n] == 1)
    def _():
        oi = pl.multiple_of(oi_ref[n] * CH, 8)
        o = oacc_ref[pl.ds(oi, CH), :] + b2_ref[...]
        o = o - jnp.max(o, axis=1, keepdims=True)
        out_ref[...] = o - jnp.log(jnp.sum(jnp.exp(o), axis=1, keepdims=True))


def _schedule(N):
    K = -(-N // CH)
    row_starts = [CH * r for r in range(K - 1)] + [N - CH]
    col_starts = [CH * c for c in range(K)]
    WL = (N - CH * (K - 1)) // 128 * 128
    TW = N - CH * (K - 1) - WL
    OV = CH * (K - 1) - (N - CH)
    steps = []
    for r in range(K):
        order = [c for c in range(K) if c != r] + [r]
        thr = OV if r == K - 1 else 0
        last_row = r == K - 1 and TW > 0
        for i, c in enumerate(order):
            diag = c == r
            steps.append((row_starts[r], col_starts[c], 1, int(c == K - 1),
                          int(i == 0), 1, int(c < r),
                          int(diag and not last_row),
                          int(diag and last_row), 0, 0, thr))
    for r in range(K - 1):
        for c in range(r + 1, K):
            steps.append((row_starts[r], col_starts[c], 1, int(c == K - 1),
                          0, 0, 1, 0, 0, int(c == K - 1), r, 0))
    steps.append((row_starts[0], col_starts[0], 0, 0, 0, 0, 0, 0, 0, 1,
                  K - 1, 0))
    cols = [np.asarray(col, dtype=np.int32) for col in zip(*steps)]
    for i in (0, 1, 2, 3):
        cols[i] = np.concatenate([cols[i], cols[i][-1:] * 0])
    return cols, len(steps), K, WL, TW


@jax.jit
def kernel(x, adj, W1, b1, W2, b2):
    N, F = x.shape
    H = W1.shape[1]
    C = W2.shape[1]
    sched, n_steps, K, WL, TW = _schedule(N)
    Npad = CH * K
    TWp = TW if TW else 8

    if TW:
        adj_tail = adj[:, N - TW:].astype(jnp.bfloat16)
    else:
        adj_tail = jnp.zeros((N, TWp), jnp.bfloat16)

    grid_spec = pltpu.PrefetchScalarGridSpec(
        num_scalar_prefetch=12,
        grid=(n_steps,),
        in_specs=[
            pl.BlockSpec((N, F), lambda n, *s: (0, 0)),
            pl.BlockSpec(memory_space=pltpu.MemorySpace.HBM),
            pl.BlockSpec((N, TWp), lambda n, *s: (0, 0)),
            pl.BlockSpec((F, H), lambda n, *s: (0, 0)),
            pl.BlockSpec((H, C), lambda n, *s: (0, 0)),
            pl.BlockSpec((1, H), lambda n, *s: (0, 0)),
            pl.BlockSpec((1, C), lambda n, *s: (0, 0)),
        ],
        out_specs=pl.BlockSpec((CH, C), lambda n, *s: (s[10][n], 0)),
        scratch_shapes=[
            pltpu.VMEM((Npad, H + C), jnp.float32),
            pltpu.VMEM((TWp, H), jnp.bfloat16),
            pltpu.VMEM((Npad, C), jnp.float32),
            pltpu.VMEM((CH, H), jnp.float32),
            pltpu.VMEM((2, CH, CH), jnp.float32),
            pltpu.SemaphoreType.DMA((2, 2)),
        ],
    )

    zero_buf = bool(sched[3][0] or sched[3][1])

    out = pl.pallas_call(
        functools.partial(_body, n_steps=n_steps, WL=WL, TW=TW,
                          zero_buf=zero_buf, N=N, Npad=Npad, H=H),
        grid_spec=grid_spec,
        out_shape=jax.ShapeDtypeStruct((N, C), jnp.float32),
    )(*sched, x, adj, adj_tail, W1, W2, b1.reshape(1, H), b2.reshape(1, C))
    return out
